# baseline (speedup 1.0000x reference)
import dataclasses

import jax
import jax.numpy as jnp
from jax import lax
from jax.experimental import pallas as pl
from jax.experimental.pallas import tpu as pltpu
from jax.experimental.pallas import tpu_sc as plsc

N = 10000
D = 128
G = 64
NCLS = 64
NC, NS = 2, 16
NW = NC * NS
CH = 64
NBUF = 3
NP = 10112
RB = NP
NBLK = NP // RB

_HI = lax.Precision.HIGHEST

_mesh = plsc.VectorSubcoreMesh(core_axis_name="c", subcore_axis_name="s")

_sc_params = pltpu.CompilerParams()
if "needs_layout_passes" in pltpu.CompilerParams.__dataclass_fields__:
    _sc_params = dataclasses.replace(_sc_params, needs_layout_passes=False)


def _deg_body(col_hbm, out_hbm, acc, cbuf):
    core = lax.axis_index("c")
    sub = lax.axis_index("s")
    wid = core * NS + sub
    nch = col_hbm.shape[1]
    zeros16 = jnp.zeros((16,), jnp.float32)
    ones16 = jnp.ones((16,), jnp.float32)

    @pl.loop(0, NP // 16)
    def _(i):
        acc[pl.ds(i * 16, 16)] = zeros16

    pltpu.sync_copy(col_hbm.at[wid], cbuf)

    @pl.loop(0, nch)
    def _(c):
        for j in range(CH // 16):
            idx = cbuf[c, pl.ds(j * 16, 16)]
            plsc.addupdate_scatter(acc, [idx], ones16)

    pltpu.sync_copy(acc, out_hbm.at[wid])


def _deg(col3):
    nch = col3.shape[1]
    k = pl.kernel(
        _deg_body,
        out_type=jax.ShapeDtypeStruct((NW, NP), jnp.float32),
        mesh=_mesh,
        compiler_params=_sc_params,
        scratch_types=[
            pltpu.VMEM((NP,), jnp.float32),
            pltpu.VMEM((nch, CH), jnp.int32),
        ],
    )
    return k(col3)


def _agg_body(y_hbm, row_hbm, col_hbm, out_hbm, acc, cbuf, rb2,
              mb0, mb1, mb2, gs0, gs1, gs2, ss0, ss1, ss2, is0, is1, is2):
    core = lax.axis_index("c")
    sub = lax.axis_index("s")
    wid = core * NS + sub
    nch = col_hbm.shape[1]
    nsteps = nch // NBUF
    zeros16 = jnp.zeros((16,), jnp.float32)

    @pl.loop(0, CH)
    def _(r):
        for j in range(D // 16):
            mb0[r, pl.ds(j * 16, 16)] = zeros16

    rpt = NP // NS
    for k in range(rpt // CH):
        pltpu.sync_copy(mb0, acc.at[pl.ds(sub * rpt + k * CH, CH)])
    if rpt % CH:
        pltpu.sync_copy(mb0.at[pl.ds(0, rpt % CH)],
                        acc.at[pl.ds(sub * rpt + (rpt // CH) * CH, rpt % CH)])
    plsc.subcore_barrier()

    mbufs = (mb0, mb1, mb2)
    gsems = (gs0, gs1, gs2)
    ssems = (ss0, ss1, ss2)
    isems = (is0, is1, is2)
    pltpu.sync_copy(col_hbm.at[wid], cbuf)
    for b in range(NBUF):
        pltpu.sync_copy(row_hbm.at[wid, b], rb2.at[b])
        pltpu.make_async_copy(y_hbm.at[rb2.at[b]], mbufs[b], gsems[b]).start()

    @pl.loop(0, nsteps)
    def _(i):
        c0 = i * NBUF
        scat = []
        for b in range(NBUF):
            pltpu.make_async_copy(y_hbm.at[rb2.at[b]], mbufs[b],
                                  gsems[b]).wait()
            scat.append(pltpu.async_copy(mbufs[b], acc.at[cbuf.at[c0 + b]],
                                         ssems[b], add=True))

            @pl.when(i < nsteps - 1)
            def _():
                pltpu.make_async_copy(row_hbm.at[wid, c0 + b + NBUF],
                                      rb2.at[b], isems[b]).start()

        for b in range(NBUF):
            scat[b].wait()

            @pl.when(i < nsteps - 1)
            def _():
                pltpu.make_async_copy(row_hbm.at[wid, c0 + b + NBUF],
                                      rb2.at[b], isems[b]).wait()
                pltpu.make_async_copy(y_hbm.at[rb2.at[b]], mbufs[b],
                                      gsems[b]).start()

    plsc.subcore_barrier()
    pltpu.sync_copy(acc.at[pl.ds(sub * rpt, rpt)],
                    out_hbm.at[core, pl.ds(sub * rpt, rpt)])


def _agg(y, row3, col3):
    nch = row3.shape[1]
    k = pl.kernel(
        _agg_body,
        out_type=jax.ShapeDtypeStruct((NC, NP, D), jnp.float32),
        mesh=_mesh,
        scratch_types=[
            pltpu.VMEM_SHARED((NP, D), jnp.float32),
            pltpu.VMEM((nch, CH), jnp.int32),
            pltpu.VMEM((NBUF, CH), jnp.int32),
        ] + [pltpu.VMEM((CH, D), jnp.float32)] * NBUF
          + [pltpu.SemaphoreType.DMA] * (3 * NBUF),
    )
    return k(y, row3, col3)


def _prep_body(degp_ref, x_ref, w_ref, dis_ref, y_ref):
    deg = jnp.sum(degp_ref[...], axis=0) + 1.0
    dis = lax.rsqrt(deg)[:, None]
    xw = jnp.dot(x_ref[...], w_ref[...], precision=_HI,
                 preferred_element_type=jnp.float32)
    dis_ref[...] = dis
    y_ref[...] = xw * dis


def _prep(degp, xp, W1):
    return pl.pallas_call(
        _prep_body,
        grid=(NBLK,),
        in_specs=[
            pl.BlockSpec((NW, RB), lambda i: (0, i)),
            pl.BlockSpec((RB, D), lambda i: (i, 0)),
            pl.BlockSpec((D, D), lambda i: (0, 0)),
        ],
        out_specs=[
            pl.BlockSpec((RB, 1), lambda i: (i, 0)),
            pl.BlockSpec((RB, D), lambda i: (i, 0)),
        ],
        out_shape=[
            jax.ShapeDtypeStruct((NP, 1), jnp.float32),
            jax.ShapeDtypeStruct((NP, D), jnp.float32),
        ],
    )(degp, xp, W1)


def _mid_body(p_ref, y1_ref, dis_ref, b1_ref, w2_ref, y2_ref):
    t = p_ref[0] + p_ref[1] + y1_ref[...]
    h = dis_ref[...] * t + b1_ref[...]
    h = jnp.where(h >= 0, h, 0.1 * h)
    y2 = jnp.dot(h, w2_ref[...], precision=_HI,
                 preferred_element_type=jnp.float32)
    y2_ref[...] = y2 * dis_ref[...]


def _mid(p, y1, dis, b1r, W2):
    return pl.pallas_call(
        _mid_body,
        grid=(NBLK,),
        in_specs=[
            pl.BlockSpec((NC, RB, D), lambda i: (0, i, 0)),
            pl.BlockSpec((RB, D), lambda i: (i, 0)),
            pl.BlockSpec((RB, 1), lambda i: (i, 0)),
            pl.BlockSpec((1, D), lambda i: (0, 0)),
            pl.BlockSpec((D, D), lambda i: (0, 0)),
        ],
        out_specs=pl.BlockSpec((RB, D), lambda i: (i, 0)),
        out_shape=jax.ShapeDtypeStruct((NP, D), jnp.float32),
    )(p, y1, dis, b1r, W2)


def _pool_body(q_ref, y2_ref, dis_ref, b2_ref, batch_ref, wl_ref, bl_ref,
               o_ref, s_acc, c_acc):
    i = pl.program_id(0)
    t = q_ref[0] + q_ref[1] + y2_ref[...]
    h2 = dis_ref[...] * t + b2_ref[...]
    h2 = jnp.where(h2 >= 0, h2, 0.1 * h2)
    b = batch_ref[0, 0, :]
    oh = (lax.broadcasted_iota(jnp.int32, (G, RB), 0) == b[None, :]).astype(
        jnp.float32)
    s_blk = jnp.dot(oh, h2, precision=_HI, preferred_element_type=jnp.float32)
    c_blk = jnp.sum(oh, axis=1, keepdims=True)

    @pl.when(i == 0)
    def _():
        s_acc[...] = jnp.zeros_like(s_acc)
        c_acc[...] = jnp.zeros_like(c_acc)

    s_acc[...] += s_blk
    c_acc[...] += c_blk

    @pl.when(i == NBLK - 1)
    def _():
        pooled = s_acc[...] / jnp.maximum(c_acc[...], 1.0)
        out = lax.dot_general(pooled, wl_ref[...], (((1,), (1,)), ((), ())),
                              precision=_HI,
                              preferred_element_type=jnp.float32)
        o_ref[...] = out + bl_ref[...]


def _pool(q, y2, dis, b2r, batch3, W_lin, blr):
    return pl.pallas_call(
        _pool_body,
        grid=(NBLK,),
        in_specs=[
            pl.BlockSpec((NC, RB, D), lambda i: (0, i, 0)),
            pl.BlockSpec((RB, D), lambda i: (i, 0)),
            pl.BlockSpec((RB, 1), lambda i: (i, 0)),
            pl.BlockSpec((1, D), lambda i: (0, 0)),
            pl.BlockSpec((1, 1, RB), lambda i: (i, 0, 0)),
            pl.BlockSpec((NCLS, D), lambda i: (0, 0)),
            pl.BlockSpec((1, NCLS), lambda i: (0, 0)),
        ],
        out_specs=pl.BlockSpec((G, NCLS), lambda i: (0, 0)),
        out_shape=jax.ShapeDtypeStruct((G, NCLS), jnp.float32),
        scratch_shapes=[
            pltpu.VMEM((G, D), jnp.float32),
            pltpu.VMEM((G, 1), jnp.float32),
        ],
    )(q, y2, dis, b2r, batch3, W_lin, blr)


def kernel(x, edge_index, batch, W1, b1, W2, b2, W_lin, b_lin):
    row = edge_index[0]
    col = edge_index[1]
    E = row.shape[0]
    estep = NW * CH * NBUF
    e_pad = -(-E // estep) * estep
    pad = e_pad - E
    ar = jnp.arange(pad, dtype=jnp.int32)
    row_p = jnp.concatenate([row, (ar * 131) % N])
    col_p = jnp.concatenate([col, N + (ar % (NP - N))])
    nch = e_pad // (NW * CH)
    row3 = row_p.reshape(NW, nch, CH)
    col3 = col_p.reshape(NW, nch, CH)
    xp = jnp.concatenate([x, jnp.zeros((NP - N, D), jnp.float32)])
    batch3 = jnp.concatenate(
        [batch, jnp.full((NP - N,), G, jnp.int32)]).reshape(NBLK, 1, RB)

    degp = _deg(col3)
    dis, y1 = _prep(degp, xp, W1)
    p = _agg(y1, row3, col3)
    y2 = _mid(p, y1, dis, b1.reshape(1, D), W2)
    q = _agg(y2, row3, col3)
    out = _pool(q, y2, dis, b2.reshape(1, D), batch3,
                W_lin, b_lin.reshape(1, NCLS))
    return out


# speedup vs baseline: 1.0786x; 1.0786x over previous
import dataclasses

import jax
import jax.numpy as jnp
from jax import lax
from jax.experimental import pallas as pl
from jax.experimental.pallas import tpu as pltpu
from jax.experimental.pallas import tpu_sc as plsc

N = 10000
D = 128
G = 64
NCLS = 64
NC, NS = 2, 16
NW = NC * NS
CH = 64
NBUF = 4
NP = 10240
RB = 2048
NBLK = NP // RB

_HI = lax.Precision.HIGHEST

_mesh = plsc.VectorSubcoreMesh(core_axis_name="c", subcore_axis_name="s")

_sc_params = pltpu.CompilerParams()
if "needs_layout_passes" in pltpu.CompilerParams.__dataclass_fields__:
    _sc_params = dataclasses.replace(_sc_params, needs_layout_passes=False)


def _deg_body(col_hbm, out_hbm, acc, cbuf):
    core = lax.axis_index("c")
    sub = lax.axis_index("s")
    wid = core * NS + sub
    nch = col_hbm.shape[1]
    zeros16 = jnp.zeros((16,), jnp.float32)
    ones16 = jnp.ones((16,), jnp.float32)

    @pl.loop(0, NP // 16)
    def _(i):
        acc[pl.ds(i * 16, 16)] = zeros16

    pltpu.sync_copy(col_hbm.at[wid], cbuf)

    @pl.loop(0, nch)
    def _(c):
        for j in range(CH // 16):
            idx = cbuf[c, pl.ds(j * 16, 16)]
            plsc.addupdate_scatter(acc, [idx], ones16)

    pltpu.sync_copy(acc, out_hbm.at[wid])


def _deg(col3):
    nch = col3.shape[1]
    k = pl.kernel(
        _deg_body,
        out_type=jax.ShapeDtypeStruct((NW, NP), jnp.float32),
        mesh=_mesh,
        compiler_params=_sc_params,
        scratch_types=[
            pltpu.VMEM((NP,), jnp.float32),
            pltpu.VMEM((nch, CH), jnp.int32),
        ],
    )
    return k(col3)


def _agg_body(y_hbm, row_hbm, col_hbm, out_hbm, acc, cb4, rb4,
              mb0, mb1, mb2, mb3,
              gs0, gs1, gs2, gs3, ss0, ss1, ss2, ss3,
              rs0, rs1, rs2, rs3, cs0, cs1, cs2, cs3):
    core = lax.axis_index("c")
    sub = lax.axis_index("s")
    wid = core * NS + sub
    nch = col_hbm.shape[1]
    nsteps = nch // NBUF
    zeros16 = jnp.zeros((16,), jnp.float32)

    @pl.loop(0, CH)
    def _(r):
        for j in range(D // 16):
            mb0[r, pl.ds(j * 16, 16)] = zeros16

    rpt = NP // NS
    for k in range(rpt // CH):
        pltpu.sync_copy(mb0, acc.at[pl.ds(sub * rpt + k * CH, CH)])
    if rpt % CH:
        pltpu.sync_copy(mb0.at[pl.ds(0, rpt % CH)],
                        acc.at[pl.ds(sub * rpt + (rpt // CH) * CH, rpt % CH)])
    plsc.subcore_barrier()

    mbufs = (mb0, mb1, mb2, mb3)
    gsems = (gs0, gs1, gs2, gs3)
    ssems = (ss0, ss1, ss2, ss3)
    rsems = (rs0, rs1, rs2, rs3)
    csems = (cs0, cs1, cs2, cs3)
    for b in range(NBUF):
        pltpu.make_async_copy(col_hbm.at[wid, b], cb4.at[b], csems[b]).start()
        pltpu.sync_copy(row_hbm.at[wid, b], rb4.at[b])
        pltpu.make_async_copy(y_hbm.at[rb4.at[b]], mbufs[b], gsems[b]).start()

    @pl.loop(0, nsteps)
    def _(i):
        c0 = i * NBUF
        scat = []
        for b in range(NBUF):
            pltpu.make_async_copy(y_hbm.at[rb4.at[b]], mbufs[b],
                                  gsems[b]).wait()
            pltpu.make_async_copy(col_hbm.at[wid, c0 + b], cb4.at[b],
                                  csems[b]).wait()
            scat.append(pltpu.async_copy(mbufs[b], acc.at[cb4.at[b]],
                                         ssems[b], add=True))

            @pl.when(i < nsteps - 1)
            def _():
                pltpu.make_async_copy(row_hbm.at[wid, c0 + b + NBUF],
                                      rb4.at[b], rsems[b]).start()

        for b in range(NBUF):
            scat[b].wait()

            @pl.when(i < nsteps - 1)
            def _():
                pltpu.make_async_copy(col_hbm.at[wid, c0 + b + NBUF],
                                      cb4.at[b], csems[b]).start()
                pltpu.make_async_copy(row_hbm.at[wid, c0 + b + NBUF],
                                      rb4.at[b], rsems[b]).wait()
                pltpu.make_async_copy(y_hbm.at[rb4.at[b]], mbufs[b],
                                      gsems[b]).start()

    plsc.subcore_barrier()
    pltpu.sync_copy(acc.at[pl.ds(sub * rpt, rpt)],
                    out_hbm.at[core, pl.ds(sub * rpt, rpt)])


def _agg(y, row3, col3):
    nch = row3.shape[1]
    k = pl.kernel(
        _agg_body,
        out_type=jax.ShapeDtypeStruct((NC, NP, D), jnp.float32),
        mesh=_mesh,
        scratch_types=[
            pltpu.VMEM_SHARED((NP, D), jnp.float32),
            pltpu.VMEM((NBUF, CH), jnp.int32),
            pltpu.VMEM((NBUF, CH), jnp.int32),
        ] + [pltpu.VMEM((CH, D), jnp.float32)] * NBUF
          + [pltpu.SemaphoreType.DMA] * (4 * NBUF),
    )
    return k(y, row3, col3)


def _prep_body(degp_ref, x_ref, w_ref, dis_ref, y_ref):
    deg = jnp.sum(degp_ref[...], axis=0) + 1.0
    dis = lax.rsqrt(deg)[:, None]
    xw = jnp.dot(x_ref[...], w_ref[...], precision=_HI,
                 preferred_element_type=jnp.float32)
    dis_ref[...] = dis
    y_ref[...] = xw * dis


def _prep(degp, xp, W1):
    return pl.pallas_call(
        _prep_body,
        grid=(NBLK,),
        in_specs=[
            pl.BlockSpec((NW, RB), lambda i: (0, i)),
            pl.BlockSpec((RB, D), lambda i: (i, 0)),
            pl.BlockSpec((D, D), lambda i: (0, 0)),
        ],
        out_specs=[
            pl.BlockSpec((RB, 1), lambda i: (i, 0)),
            pl.BlockSpec((RB, D), lambda i: (i, 0)),
        ],
        out_shape=[
            jax.ShapeDtypeStruct((NP, 1), jnp.float32),
            jax.ShapeDtypeStruct((NP, D), jnp.float32),
        ],
    )(degp, xp, W1)


def _mid_body(p_ref, y1_ref, dis_ref, b1_ref, w2_ref, y2_ref):
    t = p_ref[0] + p_ref[1] + y1_ref[...]
    h = dis_ref[...] * t + b1_ref[...]
    h = jnp.where(h >= 0, h, 0.1 * h)
    y2 = jnp.dot(h, w2_ref[...], precision=_HI,
                 preferred_element_type=jnp.float32)
    y2_ref[...] = y2 * dis_ref[...]


def _mid(p, y1, dis, b1r, W2):
    return pl.pallas_call(
        _mid_body,
        grid=(NBLK,),
        in_specs=[
            pl.BlockSpec((NC, RB, D), lambda i: (0, i, 0)),
            pl.BlockSpec((RB, D), lambda i: (i, 0)),
            pl.BlockSpec((RB, 1), lambda i: (i, 0)),
            pl.BlockSpec((1, D), lambda i: (0, 0)),
            pl.BlockSpec((D, D), lambda i: (0, 0)),
        ],
        out_specs=pl.BlockSpec((RB, D), lambda i: (i, 0)),
        out_shape=jax.ShapeDtypeStruct((NP, D), jnp.float32),
    )(p, y1, dis, b1r, W2)


def _pool_body(q_ref, y2_ref, dis_ref, b2_ref, batch_ref, wl_ref, bl_ref,
               o_ref, s_acc, c_acc):
    i = pl.program_id(0)
    t = q_ref[0] + q_ref[1] + y2_ref[...]
    h2 = dis_ref[...] * t + b2_ref[...]
    h2 = jnp.where(h2 >= 0, h2, 0.1 * h2)
    b = batch_ref[0, 0, :]
    oh = (lax.broadcasted_iota(jnp.int32, (G, RB), 0) == b[None, :]).astype(
        jnp.float32)
    s_blk = jnp.dot(oh, h2, precision=_HI, preferred_element_type=jnp.float32)
    c_blk = jnp.sum(oh, axis=1, keepdims=True)

    @pl.when(i == 0)
    def _():
        s_acc[...] = jnp.zeros_like(s_acc)
        c_acc[...] = jnp.zeros_like(c_acc)

    s_acc[...] += s_blk
    c_acc[...] += c_blk

    @pl.when(i == NBLK - 1)
    def _():
        pooled = s_acc[...] / jnp.maximum(c_acc[...], 1.0)
        out = lax.dot_general(pooled, wl_ref[...], (((1,), (1,)), ((), ())),
                              precision=_HI,
                              preferred_element_type=jnp.float32)
        o_ref[...] = out + bl_ref[...]


def _pool(q, y2, dis, b2r, batch3, W_lin, blr):
    return pl.pallas_call(
        _pool_body,
        grid=(NBLK,),
        in_specs=[
            pl.BlockSpec((NC, RB, D), lambda i: (0, i, 0)),
            pl.BlockSpec((RB, D), lambda i: (i, 0)),
            pl.BlockSpec((RB, 1), lambda i: (i, 0)),
            pl.BlockSpec((1, D), lambda i: (0, 0)),
            pl.BlockSpec((1, 1, RB), lambda i: (i, 0, 0)),
            pl.BlockSpec((NCLS, D), lambda i: (0, 0)),
            pl.BlockSpec((1, NCLS), lambda i: (0, 0)),
        ],
        out_specs=pl.BlockSpec((G, NCLS), lambda i: (0, 0)),
        out_shape=jax.ShapeDtypeStruct((G, NCLS), jnp.float32),
        scratch_shapes=[
            pltpu.VMEM((G, D), jnp.float32),
            pltpu.VMEM((G, 1), jnp.float32),
        ],
    )(q, y2, dis, b2r, batch3, W_lin, blr)


def kernel(x, edge_index, batch, W1, b1, W2, b2, W_lin, b_lin):
    row = edge_index[0]
    col = edge_index[1]
    E = row.shape[0]
    estep = NW * CH * NBUF
    e_pad = -(-E // estep) * estep
    pad = e_pad - E
    ar = jnp.arange(pad, dtype=jnp.int32)
    row_p = jnp.concatenate([row, (ar * 131) % N])
    col_p = jnp.concatenate([col, N + (ar % (NP - N))])
    nch = e_pad // (NW * CH)
    row3 = row_p.reshape(NW, nch, CH)
    col3 = col_p.reshape(NW, nch, CH)
    xp = jnp.concatenate([x, jnp.zeros((NP - N, D), jnp.float32)])
    batch3 = jnp.concatenate(
        [batch, jnp.full((NP - N,), G, jnp.int32)]).reshape(NBLK, 1, RB)

    degp = _deg(col3)
    dis, y1 = _prep(degp, xp, W1)
    p = _agg(y1, row3, col3)
    y2 = _mid(p, y1, dis, b1.reshape(1, D), W2)
    q = _agg(y2, row3, col3)
    out = _pool(q, y2, dis, b2.reshape(1, D), batch3,
                W_lin, b_lin.reshape(1, NCLS))
    return out
